# baseline (speedup 1.0000x reference)
import functools

import jax
import jax.numpy as jnp
from jax import lax
from jax.experimental import pallas as pl
from jax.experimental.pallas import tpu as pltpu
from jax.experimental.pallas import tpu_sc as plsc

_EMB = 16
_ROWS = 1000000
_BATCH = 16384
_NC = 2
_NS = 16
_NW = _NC * _NS
_BPW = _BATCH // _NW
_NBLK = _BATCH // 128
_BLKW = _BPW // 128

_mesh = plsc.VectorSubcoreMesh(core_axis_name="c", subcore_axis_name="s")
_sc_params = pltpu.CompilerParams(use_tc_tiling_on_sc=False)


def _fill_index_list(uid_v, idx_v):
    for k in range(_EMB):
        for j0 in range(0, _BPW, 16):
            vec = uid_v[pl.ds(j0, 16)] + jnp.int32(k * _ROWS)
            idx_v[k, j0 // 128, pl.ds(j0 % 128, 16)] = vec


@functools.partial(
    pl.kernel,
    out_type=(
        jax.ShapeDtypeStruct((_EMB, _NBLK, 128), jnp.float32),
        jax.ShapeDtypeStruct((_EMB, _NBLK, 128), jnp.float32),
    ),
    mesh=_mesh,
    compiler_params=_sc_params,
    scratch_types=[
        pltpu.VMEM((_BPW,), jnp.int32),
        pltpu.VMEM((_BPW,), jnp.int32),
        pltpu.VMEM((_EMB, _BLKW, 128), jnp.int32),
        pltpu.VMEM((_EMB, _BLKW, 128), jnp.int32),
        pltpu.VMEM((_EMB, _BLKW, 128), jnp.float32),
        pltpu.VMEM((_EMB, _BLKW, 128), jnp.float32),
        pltpu.SemaphoreType.DMA,
        pltpu.SemaphoreType.DMA,
    ],
)
def _gather_sc(users_h, items_h, umem_h, imem_h, uout_h, iout_h,
               uid_v, iid_v, uix_v, iix_v, urow_v, irow_v, sem_u, sem_i):
    wid = lax.axis_index("s") * _NC + lax.axis_index("c")
    base = wid * _BPW
    pltpu.sync_copy(users_h.at[pl.ds(base, _BPW)], uid_v)
    pltpu.sync_copy(items_h.at[pl.ds(base, _BPW)], iid_v)
    _fill_index_list(uid_v, uix_v)
    _fill_index_list(iid_v, iix_v)
    copies = []
    for k in range(_EMB):
        for b in range(_BLKW):
            copies.append(pltpu.async_copy(
                umem_h.at[uix_v.at[k, b]], urow_v.at[k, b], sem_u))
            copies.append(pltpu.async_copy(
                imem_h.at[iix_v.at[k, b]], irow_v.at[k, b], sem_i))
    for c in copies:
        c.wait()
    pltpu.sync_copy(urow_v, uout_h.at[:, pl.ds(wid * _BLKW, _BLKW), :])
    pltpu.sync_copy(irow_v, iout_h.at[:, pl.ds(wid * _BLKW, _BLKW), :])


@functools.partial(
    pl.kernel,
    out_type=(),
    mesh=_mesh,
    compiler_params=_sc_params,
    scratch_types=[
        pltpu.VMEM((_BPW,), jnp.int32),
        pltpu.VMEM((_BPW,), jnp.int32),
        pltpu.VMEM((_EMB, _BLKW, 128), jnp.int32),
        pltpu.VMEM((_EMB, _BLKW, 128), jnp.int32),
        pltpu.VMEM((_EMB, _BLKW, 128), jnp.float32),
        pltpu.VMEM((_EMB, _BLKW, 128), jnp.float32),
        pltpu.SemaphoreType.DMA,
        pltpu.SemaphoreType.DMA,
    ],
)
def _scatter_sc(users_h, items_h, uupd_h, iupd_h, numem_ref, nimem_ref,
                uid_v, iid_v, uix_v, iix_v, urow_v, irow_v, sem_u, sem_i):
    wid = lax.axis_index("s") * _NC + lax.axis_index("c")
    base = wid * _BPW
    pltpu.sync_copy(users_h.at[pl.ds(base, _BPW)], uid_v)
    pltpu.sync_copy(items_h.at[pl.ds(base, _BPW)], iid_v)
    pltpu.sync_copy(uupd_h.at[:, pl.ds(wid * _BLKW, _BLKW), :], urow_v)
    pltpu.sync_copy(iupd_h.at[:, pl.ds(wid * _BLKW, _BLKW), :], irow_v)
    _fill_index_list(uid_v, uix_v)
    _fill_index_list(iid_v, iix_v)
    copies = []
    for k in range(_EMB):
        for b in range(_BLKW):
            copies.append(pltpu.async_copy(
                urow_v.at[k, b], numem_ref.at[uix_v.at[k, b]], sem_u))
            copies.append(pltpu.async_copy(
                irow_v.at[k, b], nimem_ref.at[iix_v.at[k, b]], sem_i))
    for c in copies:
        c.wait()


def _gru_tc_body(ue_ref, ie_ref, w_ref, b_ref, hn_ref, ou_ref, oi_ref):
    ue = ue_ref[...]
    ie = ie_ref[...]
    w = w_ref[...]
    b = b_ref[...]
    hn = hn_ref[...]
    gu = jnp.dot(w, jnp.concatenate([ue, ie], axis=0),
                 preferred_element_type=jnp.float32) + b
    gi = jnp.dot(w, jnp.concatenate([ie, ue], axis=0),
                 preferred_element_type=jnp.float32) + b
    for g, o_ref in ((gu, ou_ref), (gi, oi_ref)):
        r = jax.nn.sigmoid(g[:_EMB, :])
        z = jax.nn.sigmoid(g[_EMB:2 * _EMB, :])
        n = jnp.tanh(g[2 * _EMB:, :] + r * hn)
        o_ref[...] = (1.0 - z) * n


_gru_tc = pl.pallas_call(
    _gru_tc_body,
    out_shape=(
        jax.ShapeDtypeStruct((_EMB, _BATCH), jnp.float32),
        jax.ShapeDtypeStruct((_EMB, _BATCH), jnp.float32),
    ),
)


def kernel(users, items, user_memory, item_memory, weight_ih, weight_hh,
           bias_ih, bias_hh):
    del weight_hh
    um_flat = user_memory.T.reshape(_EMB * _ROWS)
    im_flat = item_memory.T.reshape(_EMB * _ROWS)
    bias = (bias_ih + jnp.concatenate([bias_hh[:2 * _EMB],
                                       jnp.zeros((_EMB,), jnp.float32)])
            ).reshape(3 * _EMB, 1)
    hn = bias_hh[2 * _EMB:].reshape(_EMB, 1)

    ue3, ie3 = _gather_sc(users, items, um_flat, im_flat)
    updt_u, updt_i = _gru_tc(ue3.reshape(_EMB, _BATCH),
                             ie3.reshape(_EMB, _BATCH),
                             weight_ih, bias, hn)

    numem = jax.new_ref(um_flat)
    nimem = jax.new_ref(im_flat)
    _scatter_sc(users, items, updt_u.reshape(_EMB, _NBLK, 128),
                updt_i.reshape(_EMB, _NBLK, 128), numem, nimem)
    new_um = numem[...].reshape(_EMB, _ROWS).T
    new_im = nimem[...].reshape(_EMB, _ROWS).T
    return updt_u.T, updt_i.T, new_um, new_im


# speedup vs baseline: 2.4924x; 2.4924x over previous
import functools

import jax
import jax.numpy as jnp
from jax import lax
from jax.experimental import pallas as pl
from jax.experimental.pallas import tpu as pltpu
from jax.experimental.pallas import tpu_sc as plsc

_EMB = 16
_ROWS = 1000000
_BATCH = 16384
_NC = 2
_NS = 16
_NW = _NC * _NS
_BPW = _BATCH // _NW
_CHUNK = 128
_CPT = _BPW // _CHUNK
_NCHUNKS = _BATCH // _CHUNK

_CBLKS = (_ROWS + 127) // 128
_FULL_CBLKS = _ROWS // 128
_TAIL = _ROWS - _FULL_CBLKS * 128
_BASE_PER_W = _CBLKS // _NW
_EXTRA_W = _CBLKS - _BASE_PER_W * _NW

_mesh = plsc.VectorSubcoreMesh(core_axis_name="c", subcore_axis_name="s")
_scF = pltpu.CompilerParams(use_tc_tiling_on_sc=False)
_scT = pltpu.CompilerParams(use_tc_tiling_on_sc=True, needs_layout_passes=False)


def _wid():
    return lax.axis_index("s") * _NC + lax.axis_index("c")


def _strips_for(wid):
    return jnp.int32(_BASE_PER_W) + (wid < _EXTRA_W).astype(jnp.int32)


def _transpose_strip_in(ibuf, obuf, ncols):
    rows = lax.iota(jnp.int32, 16)
    for j in range(ncols):
        col = jnp.full((16,), j, jnp.int32)
        obuf[pl.ds(j * _EMB, _EMB)] = plsc.load_gather(ibuf, [rows, col])


def _transpose_strip_out(ibuf, obuf, ncols):
    step = lax.iota(jnp.int32, 16) * _EMB
    for k in range(_EMB):
        for j0 in range(0, ncols, 16):
            idx = step + jnp.int32(j0 * _EMB + k)
            obuf[k, pl.ds(j0, 16)] = plsc.load_gather(ibuf, [idx])


@functools.partial(
    pl.kernel,
    out_type=(
        jax.ShapeDtypeStruct((_ROWS * _EMB,), jnp.float32),
        jax.ShapeDtypeStruct((_ROWS * _EMB,), jnp.float32),
    ),
    mesh=_mesh,
    compiler_params=_scT,
    scratch_types=[
        pltpu.VMEM((_EMB, 128), jnp.float32),
        pltpu.VMEM((128 * _EMB,), jnp.float32),
    ],
)
def _detile_sc(umt_h, imt_h, ulin_h, ilin_h, ibuf, obuf):
    wid = _wid()

    @pl.loop(0, _strips_for(wid))
    def _strip(i):
        c = wid + i * _NW
        c128 = c * 128

        @pl.when(c < _FULL_CBLKS)
        def _full():
            for src, dst in ((umt_h, ulin_h), (imt_h, ilin_h)):
                pltpu.sync_copy(src.at[pl.ds(0, 8), pl.ds(c128, 128)],
                                ibuf.at[pl.ds(0, 8), :])
                pltpu.sync_copy(src.at[pl.ds(8, 8), pl.ds(c128, 128)],
                                ibuf.at[pl.ds(8, 8), :])
                _transpose_strip_in(ibuf, obuf, 128)
                pltpu.sync_copy(obuf, dst.at[pl.ds(c128 * _EMB, 128 * _EMB)])

        @pl.when(c == _FULL_CBLKS)
        def _tail():
            for src, dst in ((umt_h, ulin_h), (imt_h, ilin_h)):
                pltpu.sync_copy(src.at[pl.ds(0, 8), pl.ds(c128, _TAIL)],
                                ibuf.at[pl.ds(0, 8), pl.ds(0, _TAIL)])
                pltpu.sync_copy(src.at[pl.ds(8, 8), pl.ds(c128, _TAIL)],
                                ibuf.at[pl.ds(8, 8), pl.ds(0, _TAIL)])
                _transpose_strip_in(ibuf, obuf, _TAIL)
                pltpu.sync_copy(obuf.at[pl.ds(0, _TAIL * _EMB)],
                                dst.at[pl.ds(c128 * _EMB, _TAIL * _EMB)])


@functools.partial(
    pl.kernel,
    out_type=(
        jax.ShapeDtypeStruct((_EMB, _ROWS), jnp.float32),
        jax.ShapeDtypeStruct((_EMB, _ROWS), jnp.float32),
    ),
    mesh=_mesh,
    compiler_params=_scT,
    scratch_types=[
        pltpu.VMEM((128 * _EMB,), jnp.float32),
        pltpu.VMEM((_EMB, 128), jnp.float32),
    ],
)
def _retile_sc(ulin_h, ilin_h, umt_h, imt_h, ibuf, obuf):
    wid = _wid()

    @pl.loop(0, _strips_for(wid))
    def _strip(i):
        c = wid + i * _NW
        c128 = c * 128

        @pl.when(c < _FULL_CBLKS)
        def _full():
            for src, dst in ((ulin_h, umt_h), (ilin_h, imt_h)):
                pltpu.sync_copy(src.at[pl.ds(c128 * _EMB, 128 * _EMB)], ibuf)
                _transpose_strip_out(ibuf, obuf, 128)
                pltpu.sync_copy(obuf.at[pl.ds(0, 8), :],
                                dst.at[pl.ds(0, 8), pl.ds(c128, 128)])
                pltpu.sync_copy(obuf.at[pl.ds(8, 8), :],
                                dst.at[pl.ds(8, 8), pl.ds(c128, 128)])

        @pl.when(c == _FULL_CBLKS)
        def _tail():
            for src, dst in ((ulin_h, umt_h), (ilin_h, imt_h)):
                pltpu.sync_copy(src.at[pl.ds(c128 * _EMB, _TAIL * _EMB)],
                                ibuf.at[pl.ds(0, _TAIL * _EMB)])
                _transpose_strip_out(ibuf, obuf, _TAIL)
                pltpu.sync_copy(obuf.at[pl.ds(0, 8), pl.ds(0, _TAIL)],
                                dst.at[pl.ds(0, 8), pl.ds(c128, _TAIL)])
                pltpu.sync_copy(obuf.at[pl.ds(8, 8), pl.ds(0, _TAIL)],
                                dst.at[pl.ds(8, 8), pl.ds(c128, _TAIL)])


@functools.partial(
    pl.kernel,
    out_type=(
        jax.ShapeDtypeStruct((_NCHUNKS, _CHUNK, _EMB), jnp.float32),
        jax.ShapeDtypeStruct((_NCHUNKS, _CHUNK, _EMB), jnp.float32),
    ),
    mesh=_mesh,
    compiler_params=_scF,
    scratch_types=[
        pltpu.VMEM((_CPT, _CHUNK), jnp.int32),
        pltpu.VMEM((_CPT, _CHUNK), jnp.int32),
        pltpu.VMEM((_CPT, _CHUNK, _EMB), jnp.float32),
        pltpu.VMEM((_CPT, _CHUNK, _EMB), jnp.float32),
        pltpu.SemaphoreType.DMA,
        pltpu.SemaphoreType.DMA,
    ],
)
def _gather_sc(users_h, items_h, umem_h, imem_h, uout_h, iout_h,
               uidx_v, iidx_v, urows_v, irows_v, sem_u, sem_i):
    wid = _wid()
    base = wid * _CPT
    pltpu.sync_copy(users_h.at[pl.ds(base, _CPT)], uidx_v)
    pltpu.sync_copy(items_h.at[pl.ds(base, _CPT)], iidx_v)
    copies = []
    for j in range(_CPT):
        copies.append(pltpu.async_copy(umem_h.at[uidx_v.at[j]], urows_v.at[j], sem_u))
        copies.append(pltpu.async_copy(imem_h.at[iidx_v.at[j]], irows_v.at[j], sem_i))
    for c in copies:
        c.wait()
    pltpu.sync_copy(urows_v, uout_h.at[pl.ds(base, _CPT)])
    pltpu.sync_copy(irows_v, iout_h.at[pl.ds(base, _CPT)])


@functools.partial(
    pl.kernel,
    out_type=(),
    mesh=_mesh,
    compiler_params=_scF,
    scratch_types=[
        pltpu.VMEM((_CPT, _CHUNK), jnp.int32),
        pltpu.VMEM((_CPT, _CHUNK), jnp.int32),
        pltpu.VMEM((_CPT, _CHUNK, _EMB), jnp.float32),
        pltpu.VMEM((_CPT, _CHUNK, _EMB), jnp.float32),
        pltpu.SemaphoreType.DMA,
        pltpu.SemaphoreType.DMA,
    ],
)
def _scatter_sc(users_h, items_h, uupd_h, iupd_h, numem_ref, nimem_ref,
                uidx_v, iidx_v, urows_v, irows_v, sem_u, sem_i):
    wid = _wid()
    base = wid * _CPT
    pltpu.sync_copy(users_h.at[pl.ds(base, _CPT)], uidx_v)
    pltpu.sync_copy(items_h.at[pl.ds(base, _CPT)], iidx_v)
    pltpu.sync_copy(uupd_h.at[pl.ds(base, _CPT)], urows_v)
    pltpu.sync_copy(iupd_h.at[pl.ds(base, _CPT)], irows_v)
    copies = []
    for j in range(_CPT):
        copies.append(pltpu.async_copy(urows_v.at[j], numem_ref.at[uidx_v.at[j]], sem_u))
        copies.append(pltpu.async_copy(irows_v.at[j], nimem_ref.at[iidx_v.at[j]], sem_i))
    for c in copies:
        c.wait()


def _gru_tc_body(ue_ref, ie_ref, w_ref, b_ref, hn_ref, ou_ref, oi_ref):
    ue = ue_ref[...]
    ie = ie_ref[...]
    w = w_ref[...]
    b = b_ref[...]
    hn = hn_ref[...]
    gu = jnp.dot(jnp.concatenate([ue, ie], axis=1), w,
                 preferred_element_type=jnp.float32) + b
    gi = jnp.dot(jnp.concatenate([ie, ue], axis=1), w,
                 preferred_element_type=jnp.float32) + b
    for g, o_ref in ((gu, ou_ref), (gi, oi_ref)):
        r = jax.nn.sigmoid(g[:, :_EMB])
        z = jax.nn.sigmoid(g[:, _EMB:2 * _EMB])
        n = jnp.tanh(g[:, 2 * _EMB:] + r * hn)
        o_ref[...] = (1.0 - z) * n


_GRU_BLK = 2048
_gru_tc = pl.pallas_call(
    _gru_tc_body,
    grid=(_BATCH // _GRU_BLK,),
    in_specs=[
        pl.BlockSpec((_GRU_BLK, _EMB), lambda i: (i, 0)),
        pl.BlockSpec((_GRU_BLK, _EMB), lambda i: (i, 0)),
        pl.BlockSpec((2 * _EMB, 3 * _EMB), lambda i: (0, 0)),
        pl.BlockSpec((1, 3 * _EMB), lambda i: (0, 0)),
        pl.BlockSpec((1, _EMB), lambda i: (0, 0)),
    ],
    out_specs=(
        pl.BlockSpec((_GRU_BLK, _EMB), lambda i: (i, 0)),
        pl.BlockSpec((_GRU_BLK, _EMB), lambda i: (i, 0)),
    ),
    out_shape=(
        jax.ShapeDtypeStruct((_BATCH, _EMB), jnp.float32),
        jax.ShapeDtypeStruct((_BATCH, _EMB), jnp.float32),
    ),
)


def kernel(users, items, user_memory, item_memory, weight_ih, weight_hh,
           bias_ih, bias_hh):
    del weight_hh
    users2 = users.reshape(_NCHUNKS, _CHUNK)
    items2 = items.reshape(_NCHUNKS, _CHUNK)
    w = weight_ih.T
    bias = (bias_ih + jnp.concatenate([bias_hh[:2 * _EMB],
                                       jnp.zeros((_EMB,), jnp.float32)])
            ).reshape(1, 3 * _EMB)
    hn = bias_hh[2 * _EMB:].reshape(1, _EMB)

    ulin, ilin = _detile_sc(user_memory.T, item_memory.T)
    ue3, ie3 = _gather_sc(users2, items2,
                          ulin.reshape(_ROWS, _EMB), ilin.reshape(_ROWS, _EMB))
    upd_u, upd_i = _gru_tc(ue3.reshape(_BATCH, _EMB), ie3.reshape(_BATCH, _EMB),
                           w, bias, hn)

    numem = jax.new_ref(ulin.reshape(_ROWS, _EMB))
    nimem = jax.new_ref(ilin.reshape(_ROWS, _EMB))
    _scatter_sc(users2, items2, upd_u.reshape(_NCHUNKS, _CHUNK, _EMB),
                upd_i.reshape(_NCHUNKS, _CHUNK, _EMB), numem, nimem)
    new_umt, new_imt = _retile_sc(numem[...].reshape(_ROWS * _EMB),
                                  nimem[...].reshape(_ROWS * _EMB))
    return upd_u, upd_i, new_umt.T, new_imt.T
